# baseline (speedup 1.0000x reference)
import functools

import jax
import jax.numpy as jnp
from jax import lax
from jax.experimental import pallas as pl
from jax.experimental.pallas import tpu as pltpu
from jax.experimental.pallas import tpu_sc as plsc

_EMBED_DIM = 50
_ROW = 128
_L = 16

_info = plsc.get_sparse_core_info()
_NC = _info.num_cores
_NS = _info.num_subcores
_NW = _NC * _NS

_CHUNK = 128


def _make_transpose(vocab: int):
    n_full = vocab // _ROW
    tail = vocab - n_full * _ROW
    per_w = n_full // _NW
    extra = n_full - per_w * _NW
    mesh = plsc.VectorSubcoreMesh(core_axis_name="c", subcore_axis_name="s")

    @functools.partial(
        pl.kernel,
        mesh=mesh,
        compiler_params=pltpu.CompilerParams(needs_layout_passes=False),
        out_type=jax.ShapeDtypeStruct((vocab, _ROW), jnp.float32),
        scratch_types=[
            pltpu.VMEM((2, _EMBED_DIM, _ROW), jnp.float32),
            pltpu.VMEM((2, _ROW, _ROW), jnp.float32),
            pltpu.SemaphoreType.DMA,
            pltpu.SemaphoreType.DMA,
        ],
    )
    def transpose_kernel(tt_hbm, tail_hbm, out_hbm, in_v, out_v, rsem, wsem):
        wid = lax.axis_index("s") * _NC + lax.axis_index("c")
        my_n = jnp.where(wid < extra, per_w + 1, per_w)
        my_first = wid * per_w + jnp.minimum(wid, extra)

        if tail:
            @pl.when(wid == _NW - 1)
            def _():
                pltpu.sync_copy(tail_hbm, out_hbm.at[pl.ds(n_full * _ROW, tail)])

        def src_off(b):
            return pl.multiple_of((my_first + b) * _ROW, _ROW)

        def fire_read(b, buf):
            pltpu.async_copy(
                tt_hbm.at[:, pl.ds(src_off(b), _ROW)],
                in_v.at[buf],
                rsem,
            )

        def transpose_block(buf):
            def col(j, carry):
                iota = lax.iota(jnp.int32, _L)
                cols = jnp.full((_L,), j, jnp.int32)
                for k in range(4):
                    rows = jnp.minimum(iota + k * _L, _EMBED_DIM - 1)
                    vals = plsc.load_gather(in_v.at[buf], [rows, cols])
                    out_v[buf, j, pl.ds(k * _L, _L)] = vals
                return carry

            lax.fori_loop(0, _ROW, col, 0)

        def wait_read(buf):
            pltpu.make_async_copy(
                tt_hbm.at[:, pl.ds(0, _ROW)], in_v.at[buf], rsem
            ).wait()

        def fire_write(b, buf):
            pltpu.async_copy(
                out_v.at[buf],
                out_hbm.at[pl.ds(src_off(b), _ROW)],
                wsem,
            )

        def wait_write(buf):
            pltpu.make_async_copy(
                out_v.at[buf], out_hbm.at[pl.ds(0, _ROW)], wsem
            ).wait()

        @pl.when(my_n > 0)
        def _():
            fire_read(0, 0)

            def body(b, carry):
                buf = lax.rem(b, 2)
                nxt = 1 - buf

                @pl.when(b + 1 < my_n)
                def _():
                    fire_read(b + 1, nxt)

                wait_read(buf)

                @pl.when(b >= 2)
                def _():
                    wait_write(buf)

                transpose_block(buf)
                fire_write(b, buf)
                return carry

            lax.fori_loop(0, my_n, body, 0)
            wait_write(0)

            @pl.when(my_n > 1)
            def _():
                wait_write(1)

    return transpose_kernel


def _make_gather(total_rows: int, vocab: int):
    rows_per_w = total_rows // _NW
    n_chunks = rows_per_w // _CHUNK
    mesh = plsc.VectorSubcoreMesh(core_axis_name="c", subcore_axis_name="s")

    @functools.partial(
        pl.kernel,
        mesh=mesh,
        out_type=jax.ShapeDtypeStruct((total_rows, _ROW), jnp.float32),
        scratch_types=[
            pltpu.VMEM((rows_per_w,), jnp.int32),
            pltpu.VMEM((_CHUNK, _ROW), jnp.float32),
            pltpu.SemaphoreType.DMA,
        ],
    )
    def gather_kernel(idx_hbm, table_hbm, out_hbm, idx_v, rows_v, gsem):
        wid = lax.axis_index("s") * _NC + lax.axis_index("c")
        base = wid * rows_per_w
        pltpu.sync_copy(idx_hbm.at[pl.ds(base, rows_per_w)], idx_v)

        def body(g, carry):
            pltpu.async_copy(
                table_hbm.at[idx_v.at[pl.ds(g * _CHUNK, _CHUNK)]],
                rows_v,
                gsem,
            ).wait()
            pltpu.sync_copy(
                rows_v,
                out_hbm.at[pl.ds(base + g * _CHUNK, _CHUNK)],
            )
            return carry

        lax.fori_loop(0, n_chunks, body, 0)

    return gather_kernel


def kernel(inputs, table):
    batch, hist = inputs.shape
    total = batch * hist
    vocab = table.shape[0]
    idx = inputs.reshape(total).astype(jnp.int32)
    n_full = vocab // _ROW
    tail_rows = jnp.pad(
        table[n_full * _ROW:], ((0, 0), (0, _ROW - _EMBED_DIM))
    )
    staged = _make_transpose(vocab)(table.T, tail_rows)
    out = _make_gather(total, vocab)(idx, staged)
    return out[:, :_EMBED_DIM].reshape(batch, hist, _EMBED_DIM)


# speedup vs baseline: 1.0003x; 1.0003x over previous
import functools

import jax
import jax.numpy as jnp
from jax import lax
from jax.experimental import pallas as pl
from jax.experimental.pallas import tpu as pltpu
from jax.experimental.pallas import tpu_sc as plsc

_EMBED_DIM = 50
_ROW = 128
_L = 16

_info = plsc.get_sparse_core_info()
_NC = _info.num_cores
_NS = _info.num_subcores
_NW = _NC * _NS

_CHUNK = 128


def _make_transpose(vocab: int):
    n_full = vocab // _ROW
    tail = vocab - n_full * _ROW
    per_w = n_full // _NW
    extra = n_full - per_w * _NW
    mesh = plsc.VectorSubcoreMesh(core_axis_name="c", subcore_axis_name="s")

    @functools.partial(
        pl.kernel,
        mesh=mesh,
        compiler_params=pltpu.CompilerParams(needs_layout_passes=False),
        out_type=jax.ShapeDtypeStruct((vocab, _ROW), jnp.float32),
        scratch_types=[
            pltpu.VMEM((2, _EMBED_DIM, _ROW), jnp.float32),
            pltpu.VMEM((2, _ROW, _ROW), jnp.float32),
            pltpu.SemaphoreType.DMA,
            pltpu.SemaphoreType.DMA,
        ],
    )
    def transpose_kernel(tt_hbm, tail_hbm, out_hbm, in_v, out_v, rsem, wsem):
        wid = lax.axis_index("s") * _NC + lax.axis_index("c")
        my_n = jnp.where(wid < extra, per_w + 1, per_w)
        my_first = wid * per_w + jnp.minimum(wid, extra)

        if tail:
            @pl.when(wid == _NW - 1)
            def _():
                pltpu.sync_copy(tail_hbm, out_hbm.at[pl.ds(n_full * _ROW, tail)])

        def src_off(b):
            return pl.multiple_of((my_first + b) * _ROW, _ROW)

        def fire_read(b, buf):
            pltpu.async_copy(
                tt_hbm.at[:, pl.ds(src_off(b), _ROW)],
                in_v.at[buf],
                rsem,
            )

        def transpose_block(buf):
            iota = lax.iota(jnp.int32, _L)
            row_ids = [jnp.minimum(iota + k * _L, _EMBED_DIM - 1)
                       for k in range(4)]

            def col(j, carry):
                cols = jnp.full((_L,), j, jnp.int32)
                for k in range(4):
                    vals = plsc.load_gather(in_v.at[buf], [row_ids[k], cols])
                    out_v[buf, j, pl.ds(k * _L, _L)] = vals
                return carry

            lax.fori_loop(0, _ROW, col, 0, unroll=8)

        def wait_read(buf):
            pltpu.make_async_copy(
                tt_hbm.at[:, pl.ds(0, _ROW)], in_v.at[buf], rsem
            ).wait()

        def fire_write(b, buf):
            pltpu.async_copy(
                out_v.at[buf],
                out_hbm.at[pl.ds(src_off(b), _ROW)],
                wsem,
            )

        def wait_write(buf):
            pltpu.make_async_copy(
                out_v.at[buf], out_hbm.at[pl.ds(0, _ROW)], wsem
            ).wait()

        @pl.when(my_n > 0)
        def _():
            fire_read(0, 0)

            def body(b, carry):
                buf = lax.rem(b, 2)
                nxt = 1 - buf

                @pl.when(b + 1 < my_n)
                def _():
                    fire_read(b + 1, nxt)

                wait_read(buf)

                @pl.when(b >= 2)
                def _():
                    wait_write(buf)

                transpose_block(buf)
                fire_write(b, buf)
                return carry

            lax.fori_loop(0, my_n, body, 0)
            wait_write(0)

            @pl.when(my_n > 1)
            def _():
                wait_write(1)

    return transpose_kernel


def _make_gather(total_rows: int, vocab: int):
    rows_per_w = total_rows // _NW
    n_chunks = rows_per_w // _CHUNK
    mesh = plsc.VectorSubcoreMesh(core_axis_name="c", subcore_axis_name="s")

    @functools.partial(
        pl.kernel,
        mesh=mesh,
        out_type=jax.ShapeDtypeStruct((total_rows, _ROW), jnp.float32),
        scratch_types=[
            pltpu.VMEM((rows_per_w,), jnp.int32),
            pltpu.VMEM((_CHUNK, _ROW), jnp.float32),
            pltpu.SemaphoreType.DMA,
        ],
    )
    def gather_kernel(idx_hbm, table_hbm, out_hbm, idx_v, rows_v, gsem):
        wid = lax.axis_index("s") * _NC + lax.axis_index("c")
        base = wid * rows_per_w
        pltpu.sync_copy(idx_hbm.at[pl.ds(base, rows_per_w)], idx_v)

        def body(g, carry):
            pltpu.async_copy(
                table_hbm.at[idx_v.at[pl.ds(g * _CHUNK, _CHUNK)]],
                rows_v,
                gsem,
            ).wait()
            pltpu.sync_copy(
                rows_v,
                out_hbm.at[pl.ds(base + g * _CHUNK, _CHUNK)],
            )
            return carry

        lax.fori_loop(0, n_chunks, body, 0)

    return gather_kernel


def kernel(inputs, table):
    batch, hist = inputs.shape
    total = batch * hist
    vocab = table.shape[0]
    idx = inputs.reshape(total).astype(jnp.int32)
    n_full = vocab // _ROW
    tail_rows = jnp.pad(
        table[n_full * _ROW:], ((0, 0), (0, _ROW - _EMBED_DIM))
    )
    staged = _make_transpose(vocab)(table.T, tail_rows)
    out = _make_gather(total, vocab)(idx, staged)
    return out[:, :_EMBED_DIM].reshape(batch, hist, _EMBED_DIM)


# speedup vs baseline: 1.0022x; 1.0018x over previous
import functools

import jax
import jax.numpy as jnp
from jax import lax
from jax.experimental import pallas as pl
from jax.experimental.pallas import tpu as pltpu
from jax.experimental.pallas import tpu_sc as plsc

_EMBED_DIM = 50
_ROW = 128
_L = 16

_info = plsc.get_sparse_core_info()
_NC = _info.num_cores
_NS = _info.num_subcores
_NW = _NC * _NS

_CHUNK = 128


def _make_transpose(vocab: int):
    n_full = vocab // _ROW
    tail = vocab - n_full * _ROW
    per_w = n_full // _NW
    extra = n_full - per_w * _NW
    mesh = plsc.VectorSubcoreMesh(core_axis_name="c", subcore_axis_name="s")

    @functools.partial(
        pl.kernel,
        mesh=mesh,
        compiler_params=pltpu.CompilerParams(needs_layout_passes=False),
        out_type=jax.ShapeDtypeStruct((vocab, _ROW), jnp.float32),
        scratch_types=[
            pltpu.VMEM((2, _EMBED_DIM, _ROW + 1), jnp.float32),
            pltpu.VMEM((2, _ROW, _ROW), jnp.float32),
            pltpu.SemaphoreType.DMA,
            pltpu.SemaphoreType.DMA,
        ],
    )
    def transpose_kernel(tt_hbm, tail_hbm, out_hbm, in_v, out_v, rsem, wsem):
        wid = lax.axis_index("s") * _NC + lax.axis_index("c")
        my_n = jnp.where(wid < extra, per_w + 1, per_w)
        my_first = wid * per_w + jnp.minimum(wid, extra)

        if tail:
            @pl.when(wid == _NW - 1)
            def _():
                pltpu.sync_copy(tail_hbm, out_hbm.at[pl.ds(n_full * _ROW, tail)])

        def src_off(b):
            return pl.multiple_of((my_first + b) * _ROW, _ROW)

        def fire_read(b, buf):
            pltpu.async_copy(
                tt_hbm.at[:, pl.ds(src_off(b), _ROW)],
                in_v.at[buf, :, pl.ds(0, _ROW)],
                rsem,
            )

        def transpose_block(buf):
            iota = lax.iota(jnp.int32, _L)
            row_ids = [jnp.minimum(iota + k * _L, _EMBED_DIM - 1)
                       for k in range(4)]

            def col(j, carry):
                cols = jnp.full((_L,), j, jnp.int32)
                for k in range(4):
                    vals = plsc.load_gather(in_v.at[buf], [row_ids[k], cols])
                    out_v[buf, j, pl.ds(k * _L, _L)] = vals
                return carry

            lax.fori_loop(0, _ROW, col, 0, unroll=8)

        def wait_read(buf):
            pltpu.make_async_copy(
                tt_hbm.at[:, pl.ds(0, _ROW)], in_v.at[buf, :, pl.ds(0, _ROW)],
                rsem,
            ).wait()

        def fire_write(b, buf):
            pltpu.async_copy(
                out_v.at[buf],
                out_hbm.at[pl.ds(src_off(b), _ROW)],
                wsem,
            )

        def wait_write(buf):
            pltpu.make_async_copy(
                out_v.at[buf], out_hbm.at[pl.ds(0, _ROW)], wsem
            ).wait()

        @pl.when(my_n > 0)
        def _():
            fire_read(0, 0)

            def body(b, carry):
                buf = lax.rem(b, 2)
                nxt = 1 - buf

                @pl.when(b + 1 < my_n)
                def _():
                    fire_read(b + 1, nxt)

                wait_read(buf)

                @pl.when(b >= 2)
                def _():
                    wait_write(buf)

                transpose_block(buf)
                fire_write(b, buf)
                return carry

            lax.fori_loop(0, my_n, body, 0)
            wait_write(0)

            @pl.when(my_n > 1)
            def _():
                wait_write(1)

    return transpose_kernel


def _make_gather(total_rows: int, vocab: int):
    rows_per_w = total_rows // _NW
    n_chunks = rows_per_w // _CHUNK
    mesh = plsc.VectorSubcoreMesh(core_axis_name="c", subcore_axis_name="s")

    @functools.partial(
        pl.kernel,
        mesh=mesh,
        out_type=jax.ShapeDtypeStruct((total_rows, _ROW), jnp.float32),
        scratch_types=[
            pltpu.VMEM((rows_per_w,), jnp.int32),
            pltpu.VMEM((_CHUNK, _ROW), jnp.float32),
            pltpu.SemaphoreType.DMA,
        ],
    )
    def gather_kernel(idx_hbm, table_hbm, out_hbm, idx_v, rows_v, gsem):
        wid = lax.axis_index("s") * _NC + lax.axis_index("c")
        base = wid * rows_per_w
        pltpu.sync_copy(idx_hbm.at[pl.ds(base, rows_per_w)], idx_v)

        def body(g, carry):
            pltpu.async_copy(
                table_hbm.at[idx_v.at[pl.ds(g * _CHUNK, _CHUNK)]],
                rows_v,
                gsem,
            ).wait()
            pltpu.sync_copy(
                rows_v,
                out_hbm.at[pl.ds(base + g * _CHUNK, _CHUNK)],
            )
            return carry

        lax.fori_loop(0, n_chunks, body, 0)

    return gather_kernel


def kernel(inputs, table):
    batch, hist = inputs.shape
    total = batch * hist
    vocab = table.shape[0]
    idx = inputs.reshape(total).astype(jnp.int32)
    n_full = vocab // _ROW
    tail_rows = jnp.pad(
        table[n_full * _ROW:], ((0, 0), (0, _ROW - _EMBED_DIM))
    )
    staged = _make_transpose(vocab)(table.T, tail_rows)
    out = _make_gather(total, vocab)(idx, staged)
    return out[:, :_EMBED_DIM].reshape(batch, hist, _EMBED_DIM)


# speedup vs baseline: 1.1611x; 1.1586x over previous
import functools

import jax
import jax.numpy as jnp
from jax import lax
from jax.experimental import pallas as pl
from jax.experimental.pallas import tpu as pltpu
from jax.experimental.pallas import tpu_sc as plsc

_EMBED_DIM = 50
_ROW = 128

_info = plsc.get_sparse_core_info()
_NC = _info.num_cores
_NS = _info.num_subcores
_NW = _NC * _NS

_CHUNK = 128
_TBLK = 512


def _transpose_block(tt_ref, out_ref):
    t = jnp.transpose(tt_ref[...], (1, 0))
    pad = jnp.zeros((_TBLK, _ROW - _EMBED_DIM), jnp.float32)
    out_ref[...] = jnp.concatenate([t, pad], axis=1)


def _make_transpose(vocab: int):
    grid = (vocab + _TBLK - 1) // _TBLK
    return pl.pallas_call(
        _transpose_block,
        grid=(grid,),
        in_specs=[
            pl.BlockSpec((_EMBED_DIM, _TBLK), lambda i: (0, i)),
        ],
        out_specs=pl.BlockSpec((_TBLK, _ROW), lambda i: (i, 0)),
        out_shape=jax.ShapeDtypeStruct((vocab, _ROW), jnp.float32),
    )


def _make_gather(total_rows: int, vocab: int):
    rows_per_w = total_rows // _NW
    n_chunks = rows_per_w // _CHUNK
    mesh = plsc.VectorSubcoreMesh(core_axis_name="c", subcore_axis_name="s")

    @functools.partial(
        pl.kernel,
        mesh=mesh,
        out_type=jax.ShapeDtypeStruct((total_rows, _ROW), jnp.float32),
        scratch_types=[
            pltpu.VMEM((rows_per_w,), jnp.int32),
            pltpu.VMEM((_CHUNK, _ROW), jnp.float32),
            pltpu.SemaphoreType.DMA,
        ],
    )
    def gather_kernel(idx_hbm, table_hbm, out_hbm, idx_v, rows_v, gsem):
        wid = lax.axis_index("s") * _NC + lax.axis_index("c")
        base = wid * rows_per_w
        pltpu.sync_copy(idx_hbm.at[pl.ds(base, rows_per_w)], idx_v)

        def body(g, carry):
            pltpu.async_copy(
                table_hbm.at[idx_v.at[pl.ds(g * _CHUNK, _CHUNK)]],
                rows_v,
                gsem,
            ).wait()
            pltpu.sync_copy(
                rows_v,
                out_hbm.at[pl.ds(base + g * _CHUNK, _CHUNK)],
            )
            return carry

        lax.fori_loop(0, n_chunks, body, 0)

    return gather_kernel


def kernel(inputs, table):
    batch, hist = inputs.shape
    total = batch * hist
    vocab = table.shape[0]
    idx = inputs.reshape(total).astype(jnp.int32)
    staged = _make_transpose(vocab)(table.T)
    out = _make_gather(total, vocab)(idx, staged)
    return out[:, :_EMBED_DIM].reshape(batch, hist, _EMBED_DIM)


# speedup vs baseline: 1.9470x; 1.6768x over previous
import functools

import jax
import jax.numpy as jnp
from jax import lax
from jax.experimental import pallas as pl
from jax.experimental.pallas import tpu as pltpu
from jax.experimental.pallas import tpu_sc as plsc

_EMBED_DIM = 50
_ROW = 128

_info = plsc.get_sparse_core_info()
_NC = _info.num_cores
_NS = _info.num_subcores
_NW = _NC * _NS

_CHUNK = 128
_TBLK = 2048


def _transpose_block(tt_ref, out_ref):
    t = jnp.transpose(tt_ref[...], (1, 0))
    pad = jnp.zeros((_TBLK, _ROW - _EMBED_DIM), jnp.float32)
    out_ref[...] = jnp.concatenate([t, pad], axis=1)


def _make_transpose(vocab: int):
    grid = (vocab + _TBLK - 1) // _TBLK
    return pl.pallas_call(
        _transpose_block,
        grid=(grid,),
        in_specs=[
            pl.BlockSpec((_EMBED_DIM, _TBLK), lambda i: (0, i)),
        ],
        out_specs=pl.BlockSpec((_TBLK, _ROW), lambda i: (i, 0)),
        out_shape=jax.ShapeDtypeStruct((vocab, _ROW), jnp.float32),
    )


def _make_gather(total_rows: int, vocab: int):
    rows_per_w = total_rows // _NW
    n_chunks = rows_per_w // _CHUNK
    mesh = plsc.VectorSubcoreMesh(core_axis_name="c", subcore_axis_name="s")

    @functools.partial(
        pl.kernel,
        mesh=mesh,
        out_type=jax.ShapeDtypeStruct((total_rows, _ROW), jnp.float32),
        scratch_types=[
            pltpu.VMEM((rows_per_w,), jnp.int32),
            pltpu.VMEM((_CHUNK, _ROW), jnp.float32),
            pltpu.SemaphoreType.DMA,
        ],
    )
    def gather_kernel(idx_hbm, table_hbm, out_hbm, idx_v, rows_v, gsem):
        wid = lax.axis_index("s") * _NC + lax.axis_index("c")
        base = wid * rows_per_w
        pltpu.sync_copy(idx_hbm.at[pl.ds(base, rows_per_w)], idx_v)

        def body(g, carry):
            pltpu.async_copy(
                table_hbm.at[idx_v.at[pl.ds(g * _CHUNK, _CHUNK)]],
                rows_v,
                gsem,
            ).wait()
            pltpu.sync_copy(
                rows_v,
                out_hbm.at[pl.ds(base + g * _CHUNK, _CHUNK)],
            )
            return carry

        lax.fori_loop(0, n_chunks, body, 0)

    return gather_kernel


def kernel(inputs, table):
    batch, hist = inputs.shape
    total = batch * hist
    vocab = table.shape[0]
    idx = inputs.reshape(total).astype(jnp.int32)
    staged = _make_transpose(vocab)(table.T)
    out = _make_gather(total, vocab)(idx, staged)
    return out[:, :_EMBED_DIM].reshape(batch, hist, _EMBED_DIM)


# speedup vs baseline: 2.3823x; 1.2236x over previous
import functools

import jax
import jax.numpy as jnp
from jax import lax
from jax.experimental import pallas as pl
from jax.experimental.pallas import tpu as pltpu
from jax.experimental.pallas import tpu_sc as plsc

_EMBED_DIM = 50
_ROW = 128

_info = plsc.get_sparse_core_info()
_NC = _info.num_cores
_NS = _info.num_subcores
_NW = _NC * _NS

_CHUNK = 128
_TBLK = 8192


def _transpose_block(tt_ref, out_ref):
    t = jnp.transpose(tt_ref[...], (1, 0))
    pad = jnp.zeros((_TBLK, _ROW - _EMBED_DIM), jnp.float32)
    out_ref[...] = jnp.concatenate([t, pad], axis=1)


def _make_transpose(vocab: int):
    grid = (vocab + _TBLK - 1) // _TBLK
    return pl.pallas_call(
        _transpose_block,
        grid=(grid,),
        in_specs=[
            pl.BlockSpec((_EMBED_DIM, _TBLK), lambda i: (0, i)),
        ],
        out_specs=pl.BlockSpec((_TBLK, _ROW), lambda i: (i, 0)),
        out_shape=jax.ShapeDtypeStruct((vocab, _ROW), jnp.float32),
    )


def _make_gather(total_rows: int, vocab: int):
    rows_per_w = total_rows // _NW
    n_chunks = rows_per_w // _CHUNK
    mesh = plsc.VectorSubcoreMesh(core_axis_name="c", subcore_axis_name="s")

    @functools.partial(
        pl.kernel,
        mesh=mesh,
        out_type=jax.ShapeDtypeStruct((total_rows, _ROW), jnp.float32),
        scratch_types=[
            pltpu.VMEM((rows_per_w,), jnp.int32),
            pltpu.VMEM((_CHUNK, _ROW), jnp.float32),
            pltpu.SemaphoreType.DMA,
        ],
    )
    def gather_kernel(idx_hbm, table_hbm, out_hbm, idx_v, rows_v, gsem):
        wid = lax.axis_index("s") * _NC + lax.axis_index("c")
        base = wid * rows_per_w
        pltpu.sync_copy(idx_hbm.at[pl.ds(base, rows_per_w)], idx_v)

        def body(g, carry):
            pltpu.async_copy(
                table_hbm.at[idx_v.at[pl.ds(g * _CHUNK, _CHUNK)]],
                rows_v,
                gsem,
            ).wait()
            pltpu.sync_copy(
                rows_v,
                out_hbm.at[pl.ds(base + g * _CHUNK, _CHUNK)],
            )
            return carry

        lax.fori_loop(0, n_chunks, body, 0)

    return gather_kernel


def kernel(inputs, table):
    batch, hist = inputs.shape
    total = batch * hist
    vocab = table.shape[0]
    idx = inputs.reshape(total).astype(jnp.int32)
    staged = _make_transpose(vocab)(table.T)
    out = _make_gather(total, vocab)(idx, staged)
    return out[:, :_EMBED_DIM].reshape(batch, hist, _EMBED_DIM)


# speedup vs baseline: 2.4499x; 1.0284x over previous
import functools

import jax
import jax.numpy as jnp
from jax import lax
from jax.experimental import pallas as pl
from jax.experimental.pallas import tpu as pltpu
from jax.experimental.pallas import tpu_sc as plsc

_EMBED_DIM = 50
_ROW = 128

_info = plsc.get_sparse_core_info()
_NC = _info.num_cores
_NS = _info.num_subcores
_NW = _NC * _NS

_CHUNK = 128
_TBLK = 16384


def _transpose_block(tt_ref, out_ref):
    t = jnp.transpose(tt_ref[...], (1, 0))
    pad = jnp.zeros((_TBLK, _ROW - _EMBED_DIM), jnp.float32)
    out_ref[...] = jnp.concatenate([t, pad], axis=1)


def _make_transpose(vocab: int):
    grid = (vocab + _TBLK - 1) // _TBLK
    return pl.pallas_call(
        _transpose_block,
        grid=(grid,),
        in_specs=[
            pl.BlockSpec((_EMBED_DIM, _TBLK), lambda i: (0, i)),
        ],
        out_specs=pl.BlockSpec((_TBLK, _ROW), lambda i: (i, 0)),
        out_shape=jax.ShapeDtypeStruct((vocab, _ROW), jnp.float32),
    )


def _make_gather(total_rows: int, vocab: int):
    rows_per_w = total_rows // _NW
    n_chunks = rows_per_w // _CHUNK
    mesh = plsc.VectorSubcoreMesh(core_axis_name="c", subcore_axis_name="s")

    @functools.partial(
        pl.kernel,
        mesh=mesh,
        out_type=jax.ShapeDtypeStruct((total_rows, _ROW), jnp.float32),
        scratch_types=[
            pltpu.VMEM((rows_per_w,), jnp.int32),
            pltpu.VMEM((_CHUNK, _ROW), jnp.float32),
            pltpu.SemaphoreType.DMA,
        ],
    )
    def gather_kernel(idx_hbm, table_hbm, out_hbm, idx_v, rows_v, gsem):
        wid = lax.axis_index("s") * _NC + lax.axis_index("c")
        base = wid * rows_per_w
        pltpu.sync_copy(idx_hbm.at[pl.ds(base, rows_per_w)], idx_v)

        def body(g, carry):
            pltpu.async_copy(
                table_hbm.at[idx_v.at[pl.ds(g * _CHUNK, _CHUNK)]],
                rows_v,
                gsem,
            ).wait()
            pltpu.sync_copy(
                rows_v,
                out_hbm.at[pl.ds(base + g * _CHUNK, _CHUNK)],
            )
            return carry

        lax.fori_loop(0, n_chunks, body, 0)

    return gather_kernel


def kernel(inputs, table):
    batch, hist = inputs.shape
    total = batch * hist
    vocab = table.shape[0]
    idx = inputs.reshape(total).astype(jnp.int32)
    staged = _make_transpose(vocab)(table.T)
    out = _make_gather(total, vocab)(idx, staged)
    return out[:, :_EMBED_DIM].reshape(batch, hist, _EMBED_DIM)


# speedup vs baseline: 2.9608x; 1.2085x over previous
import functools

import jax
import jax.numpy as jnp
from jax import lax
from jax.experimental import pallas as pl
from jax.experimental.pallas import tpu as pltpu
from jax.experimental.pallas import tpu_sc as plsc

_EMBED_DIM = 50
_ROW = 128

_info = plsc.get_sparse_core_info()
_NC = _info.num_cores
_NS = _info.num_subcores
_NW = _NC * _NS

_CHUNK = 128
_TBLK = 16384


def _transpose_block(tt_ref, out_ref):
    t = jnp.transpose(tt_ref[...], (1, 0))
    pad = jnp.zeros((_TBLK, _ROW - _EMBED_DIM), jnp.float32)
    out_ref[...] = jnp.concatenate([t, pad], axis=1)


def _make_transpose(vocab: int):
    grid = (vocab + _TBLK - 1) // _TBLK
    return pl.pallas_call(
        _transpose_block,
        grid=(grid,),
        in_specs=[
            pl.BlockSpec((_EMBED_DIM, _TBLK), lambda i: (0, i)),
        ],
        out_specs=pl.BlockSpec((_TBLK, _ROW), lambda i: (i, 0)),
        out_shape=jax.ShapeDtypeStruct((vocab, _ROW), jnp.float32),
    )


def _make_gather(total_rows: int, vocab: int):
    rows_per_w = total_rows // _NW
    n_chunks = rows_per_w // _CHUNK
    mesh = plsc.VectorSubcoreMesh(core_axis_name="c", subcore_axis_name="s")

    @functools.partial(
        pl.kernel,
        mesh=mesh,
        out_type=jax.ShapeDtypeStruct((total_rows, _ROW), jnp.float32),
        scratch_types=[
            pltpu.VMEM((rows_per_w,), jnp.int32),
            pltpu.VMEM((2, _CHUNK, _ROW), jnp.float32),
            pltpu.SemaphoreType.DMA,
            pltpu.SemaphoreType.DMA,
        ],
    )
    def gather_kernel(idx_hbm, table_hbm, out_hbm, idx_v, rows_v, gsem, wsem):
        wid = lax.axis_index("s") * _NC + lax.axis_index("c")
        base = wid * rows_per_w
        pltpu.sync_copy(idx_hbm.at[pl.ds(base, rows_per_w)], idx_v)

        def fire_gather(g, buf):
            pltpu.async_copy(
                table_hbm.at[idx_v.at[pl.ds(g * _CHUNK, _CHUNK)]],
                rows_v.at[buf],
                gsem,
            )

        def wait_gather(buf):
            pltpu.make_async_copy(
                table_hbm.at[idx_v.at[pl.ds(0, _CHUNK)]], rows_v.at[buf], gsem
            ).wait()

        def fire_write(g, buf):
            pltpu.async_copy(
                rows_v.at[buf],
                out_hbm.at[pl.ds(base + g * _CHUNK, _CHUNK)],
                wsem,
            )

        def wait_write(buf):
            pltpu.make_async_copy(
                rows_v.at[buf], out_hbm.at[pl.ds(base, _CHUNK)], wsem
            ).wait()

        fire_gather(0, 0)

        def body(g, carry):
            buf = lax.rem(g, 2)
            nxt = 1 - buf

            @pl.when(g + 1 < n_chunks)
            def _():
                @pl.when(g >= 1)
                def _():
                    wait_write(nxt)

                fire_gather(g + 1, nxt)

            wait_gather(buf)
            fire_write(g, buf)
            return carry

        lax.fori_loop(0, n_chunks, body, 0)
        wait_write(lax.rem(n_chunks - 1, 2))

    return gather_kernel


def kernel(inputs, table):
    batch, hist = inputs.shape
    total = batch * hist
    vocab = table.shape[0]
    idx = inputs.reshape(total).astype(jnp.int32)
    staged = _make_transpose(vocab)(table.T)
    out = _make_gather(total, vocab)(idx, staged)
    return out[:, :_EMBED_DIM].reshape(batch, hist, _EMBED_DIM)
